# Initial kernel scaffold; baseline (speedup 1.0000x reference)
import jax
import jax.numpy as jnp
from jax.experimental import pallas as pl


def kernel(x):
    raise NotImplementedError("write your pallas kernel here")


# speedup vs baseline: 1.8891x; 1.8891x over previous
import jax
import jax.numpy as jnp
from jax.experimental import pallas as pl
from jax.experimental.pallas import tpu as pltpu

_K = 16
_ITERS = 10


def _kmeans_step(x_ref, cl_ref, c_ref, c_scr):
    step = pl.program_id(0)

    @pl.when(step == 0)
    def _init():
        c_scr[...] = x_ref[: _K, :]

    x = x_ref[...]
    c = c_scr[...]
    n = x.shape[0]

    dcols = []
    for k in range(_K):
        diff = x - c[k, :][None, :]
        dcols.append(jnp.sum(diff * diff, axis=1))
    d = jnp.stack(dcols, axis=0)
    cl = jnp.argmin(d, axis=0).astype(jnp.int32)
    cl_ref[...] = cl

    oh = (jax.lax.broadcasted_iota(jnp.int32, (_K, n), 0)
          == cl[None, :]).astype(jnp.float32)
    sums = jax.lax.dot_general(oh, x, (((1,), (0,)), ((), ())),
                               precision=jax.lax.Precision.HIGHEST,
                               preferred_element_type=jnp.float32)
    cnt = jnp.sum(oh, axis=1, keepdims=True)
    c_new = sums / cnt
    c_scr[...] = c_new
    c_ref[...] = c_new


def kernel(x):
    n, d = x.shape
    cl, c = pl.pallas_call(
        _kmeans_step,
        grid=(_ITERS,),
        in_specs=[pl.BlockSpec((n, d), lambda i: (0, 0))],
        out_specs=[
            pl.BlockSpec((n,), lambda i: (0,)),
            pl.BlockSpec((_K, d), lambda i: (0, 0)),
        ],
        out_shape=[
            jax.ShapeDtypeStruct((n,), jnp.int32),
            jax.ShapeDtypeStruct((_K, d), jnp.float32),
        ],
        scratch_shapes=[pltpu.VMEM((_K, d), jnp.float32)],
        compiler_params=pltpu.CompilerParams(
            dimension_semantics=("arbitrary",)),
    )(x)
    return cl, c


# speedup vs baseline: 3.5562x; 1.8825x over previous
import jax
import jax.numpy as jnp
import numpy as np
from jax.experimental import pallas as pl
from jax.experimental.pallas import tpu as pltpu

_K = 16
_ITERS = 10
_B = 8192

_PERM = (np.arange(64) % 8) * 8 + np.arange(64) // 8


def _exact_dist_row(xb, ck):
    diff = xb - ck
    d2 = diff * diff
    t04 = d2[0:8, :] + d2[32:40, :]
    t26 = d2[16:24, :] + d2[48:56, :]
    t15 = d2[8:16, :] + d2[40:48, :]
    t37 = d2[24:32, :] + d2[56:64, :]
    r = (t04 + t26) + (t15 + t37)
    acc = r[0:1, :]
    for a in range(1, 8):
        acc = acc + r[a : a + 1, :]
    return acc


def _kmeans_step(xtp_ref, cl_ref, ctp_ref, ctp_scr):
    step = pl.program_id(0)

    @pl.when(step == 0)
    def _init():
        ctp_scr[...] = xtp_ref[:, : _K]

    n = xtp_ref.shape[1]
    ctp = ctp_scr[...]

    def chunk_body(i, carry):
        sums_tp, cnt = carry
        xb = xtp_ref[:, pl.ds(i * _B, _B)]
        rows = [_exact_dist_row(xb, ctp[:, k : k + 1]) for k in range(_K)]
        dist = jnp.concatenate(rows, axis=0)
        cl_b = jnp.argmin(dist, axis=0).astype(jnp.int32)
        cl_ref[pl.ds(i * _B, _B)] = cl_b
        oh = (jax.lax.broadcasted_iota(jnp.int32, (_K, _B), 0)
              == cl_b[None, :]).astype(jnp.float32)
        part = jax.lax.dot_general(xb, oh, (((1,), (1,)), ((), ())),
                                   precision=jax.lax.Precision.HIGHEST,
                                   preferred_element_type=jnp.float32)
        cnt_part = jnp.sum(oh, axis=1)[None, :]
        return sums_tp + part, cnt + cnt_part

    sums_tp, cnt = jax.lax.fori_loop(
        0, n // _B, chunk_body,
        (jnp.zeros((64, _K), jnp.float32), jnp.zeros((1, _K), jnp.float32)))

    c_new_tp = sums_tp / cnt
    ctp_scr[...] = c_new_tp
    ctp_ref[...] = c_new_tp


def kernel(x):
    n, d = x.shape
    xtp = x.T[_PERM, :]
    cl, ctp = pl.pallas_call(
        _kmeans_step,
        grid=(_ITERS,),
        in_specs=[pl.BlockSpec((d, n), lambda i: (0, 0))],
        out_specs=[
            pl.BlockSpec((n,), lambda i: (0,)),
            pl.BlockSpec((d, _K), lambda i: (0, 0)),
        ],
        out_shape=[
            jax.ShapeDtypeStruct((n,), jnp.int32),
            jax.ShapeDtypeStruct((d, _K), jnp.float32),
        ],
        scratch_shapes=[pltpu.VMEM((d, _K), jnp.float32)],
        compiler_params=pltpu.CompilerParams(
            dimension_semantics=("arbitrary",)),
    )(xtp)
    c = ctp[_PERM, :].T
    return cl, c
